# baseline (speedup 1.0000x reference)
import functools

import jax
import jax.numpy as jnp
from jax import lax
from jax.experimental import pallas as pl
from jax.experimental.pallas import tpu as pltpu
from jax.experimental.pallas import tpu_sc as plsc

NC = 2
NS = 16
NW = NC * NS


@functools.partial(jax.jit, static_argnames=("batch", "seq", "emb"))
def _embedding_gather(idx_t, table, *, batch, seq, emb):
    ipw = batch // NW
    n_pairs = seq // 2 - 1

    mesh = plsc.VectorSubcoreMesh(core_axis_name="c", subcore_axis_name="s")

    @functools.partial(
        pl.kernel,
        mesh=mesh,
        compiler_params=pltpu.CompilerParams(use_tc_tiling_on_sc=False),
        out_type=jax.ShapeDtypeStruct((batch, seq, emb), jnp.float32),
        scratch_types=[
            pltpu.VMEM((seq, ipw), jnp.int32),
            pltpu.VMEM((ipw, emb), jnp.float32),
            pltpu.VMEM((ipw, emb), jnp.float32),
            pltpu.SemaphoreType.DMA,
            pltpu.SemaphoreType.DMA,
        ],
    )
    def k(table_hbm, idxt_hbm, out_hbm, idx_v, b0, b1, g0, g1):
        wid = lax.axis_index("s") * NC + lax.axis_index("c")
        samples = pl.ds(wid * ipw, ipw)
        pltpu.sync_copy(idxt_hbm.at[:, samples], idx_v)

        pltpu.async_copy(table_hbm.at[idx_v.at[0]], b0, g0)
        pltpu.async_copy(table_hbm.at[idx_v.at[1]], b1, g1)

        def body(p, _):
            s0 = 2 * p
            pltpu.make_async_copy(table_hbm.at[idx_v.at[s0]], b0, g0).wait()
            pltpu.sync_copy(b0, out_hbm.at[samples, s0])
            pltpu.async_copy(table_hbm.at[idx_v.at[s0 + 2]], b0, g0)
            pltpu.make_async_copy(table_hbm.at[idx_v.at[s0 + 1]], b1, g1).wait()
            pltpu.sync_copy(b1, out_hbm.at[samples, s0 + 1])
            pltpu.async_copy(table_hbm.at[idx_v.at[s0 + 3]], b1, g1)
            return _

        lax.fori_loop(0, n_pairs, body, None)

        sl = seq - 2
        pltpu.make_async_copy(table_hbm.at[idx_v.at[sl]], b0, g0).wait()
        pltpu.sync_copy(b0, out_hbm.at[samples, sl])
        pltpu.make_async_copy(table_hbm.at[idx_v.at[sl + 1]], b1, g1).wait()
        pltpu.sync_copy(b1, out_hbm.at[samples, sl + 1])

    return k(table, idx_t)


def kernel(input_tensor, embedding_table):
    batch, seq = input_tensor.shape
    _, emb = embedding_table.shape
    idx_t = input_tensor.T.astype(jnp.int32)
    return _embedding_gather(idx_t, embedding_table,
                             batch=batch, seq=seq, emb=emb)


# speedup vs baseline: 1.0082x; 1.0082x over previous
import functools

import jax
import jax.numpy as jnp
from jax import lax
from jax.experimental import pallas as pl
from jax.experimental.pallas import tpu as pltpu
from jax.experimental.pallas import tpu_sc as plsc

NC = 2
NS = 16
NW = NC * NS


NBUF = 5


@functools.partial(jax.jit, static_argnames=("batch", "seq", "emb"))
def _embedding_gather(idx_t, table, *, batch, seq, emb):
    ipw = batch // NW
    n_rounds = seq // NBUF

    mesh = plsc.VectorSubcoreMesh(core_axis_name="c", subcore_axis_name="s")

    @functools.partial(
        pl.kernel,
        mesh=mesh,
        compiler_params=pltpu.CompilerParams(use_tc_tiling_on_sc=False),
        out_type=jax.ShapeDtypeStruct((batch, seq, emb), jnp.float32),
        scratch_types=[
            pltpu.VMEM((seq, ipw), jnp.int32),
        ] + [pltpu.VMEM((ipw, emb), jnp.float32)] * NBUF
          + [pltpu.SemaphoreType.DMA] * (2 * NBUF),
    )
    def k(table_hbm, idxt_hbm, out_hbm, idx_v, *bufs_and_sems):
        bufs = bufs_and_sems[:NBUF]
        gsem = bufs_and_sems[NBUF:2 * NBUF]
        wsem = bufs_and_sems[2 * NBUF:]
        wid = lax.axis_index("s") * NC + lax.axis_index("c")
        samples = pl.ds(wid * ipw, ipw)
        pltpu.sync_copy(idxt_hbm.at[:, samples], idx_v)

        for b in range(NBUF):
            pltpu.async_copy(table_hbm.at[idx_v.at[b]], bufs[b], gsem[b])

        def body(p, _):
            s0 = p * NBUF
            for b in range(NBUF):
                pltpu.make_async_copy(
                    table_hbm.at[idx_v.at[s0 + b]], bufs[b], gsem[b]).wait()
                pltpu.async_copy(bufs[b], out_hbm.at[samples, s0 + b], wsem[b])
            for b in range(NBUF):
                pltpu.make_async_copy(
                    bufs[b], out_hbm.at[samples, s0 + b], wsem[b]).wait()
                pltpu.async_copy(
                    table_hbm.at[idx_v.at[s0 + NBUF + b]], bufs[b], gsem[b])
            return _

        lax.fori_loop(0, n_rounds - 1, body, None)

        sl = seq - NBUF
        for b in range(NBUF):
            pltpu.make_async_copy(
                table_hbm.at[idx_v.at[sl + b]], bufs[b], gsem[b]).wait()
            pltpu.async_copy(bufs[b], out_hbm.at[samples, sl + b], wsem[b])
        for b in range(NBUF):
            pltpu.make_async_copy(
                bufs[b], out_hbm.at[samples, sl + b], wsem[b]).wait()

    return k(table, idx_t)


def kernel(input_tensor, embedding_table):
    batch, seq = input_tensor.shape
    _, emb = embedding_table.shape
    idx_t = input_tensor.T.astype(jnp.int32)
    return _embedding_gather(idx_t, embedding_table,
                             batch=batch, seq=seq, emb=emb)


# speedup vs baseline: 1.0455x; 1.0371x over previous
import functools

import jax
import jax.numpy as jnp
from jax import lax
from jax.experimental import pallas as pl
from jax.experimental.pallas import tpu as pltpu
from jax.experimental.pallas import tpu_sc as plsc

NC = 2
NS = 16
NW = NC * NS


NBUF = 5


@functools.partial(jax.jit, static_argnames=("batch", "seq", "emb"))
def _embedding_gather(idx_t, table, *, batch, seq, emb):
    ipw = batch // NW
    n_rounds = seq // NBUF

    mesh = plsc.VectorSubcoreMesh(core_axis_name="c", subcore_axis_name="s")

    @functools.partial(
        pl.kernel,
        mesh=mesh,
        compiler_params=pltpu.CompilerParams(use_tc_tiling_on_sc=False),
        out_type=jax.ShapeDtypeStruct((batch, seq, emb), jnp.float32),
        scratch_types=[
            pltpu.VMEM((seq, ipw), jnp.int32),
        ] + [pltpu.VMEM((ipw, 2 * emb), jnp.float32)] * NBUF
          + [pltpu.SemaphoreType.DMA] * (2 * NBUF),
    )
    def k(table_hbm, idxt_hbm, out_hbm, idx_v, *bufs_and_sems):
        bufs = bufs_and_sems[:NBUF]
        gsem = bufs_and_sems[NBUF:2 * NBUF]
        wsem = bufs_and_sems[2 * NBUF:]
        wid = lax.axis_index("s") * NC + lax.axis_index("c")
        samples = pl.ds(wid * ipw, ipw)
        pltpu.sync_copy(idxt_hbm.at[:, samples], idx_v)

        for b in range(NBUF):
            pltpu.async_copy(table_hbm.at[idx_v.at[b]], bufs[b], gsem[b])

        emb_half = pl.ds(0, emb)

        def body(p, _):
            s0 = p * NBUF
            for b in range(NBUF):
                pltpu.make_async_copy(
                    table_hbm.at[idx_v.at[s0 + b]], bufs[b], gsem[b]).wait()
                pltpu.async_copy(bufs[b].at[:, emb_half],
                                 out_hbm.at[samples, s0 + b], wsem[b])
            for b in range(NBUF):
                pltpu.make_async_copy(
                    bufs[b].at[:, emb_half],
                    out_hbm.at[samples, s0 + b], wsem[b]).wait()
                pltpu.async_copy(
                    table_hbm.at[idx_v.at[s0 + NBUF + b]], bufs[b], gsem[b])
            return _

        lax.fori_loop(0, n_rounds - 1, body, None)

        sl = seq - NBUF
        for b in range(NBUF):
            pltpu.make_async_copy(
                table_hbm.at[idx_v.at[sl + b]], bufs[b], gsem[b]).wait()
            pltpu.async_copy(bufs[b].at[:, emb_half],
                             out_hbm.at[samples, sl + b], wsem[b])
        for b in range(NBUF):
            pltpu.make_async_copy(
                bufs[b].at[:, emb_half],
                out_hbm.at[samples, sl + b], wsem[b]).wait()

    return k(table, idx_t)


def kernel(input_tensor, embedding_table):
    batch, seq = input_tensor.shape
    _, emb = embedding_table.shape
    idx_t = input_tensor.T.astype(jnp.int32)
    table_p = jnp.pad(embedding_table, ((0, 0), (0, 128 - emb)))
    return _embedding_gather(idx_t, table_p,
                             batch=batch, seq=seq, emb=emb)


# speedup vs baseline: 1.0610x; 1.0148x over previous
import functools

import jax
import jax.numpy as jnp
from jax import lax
from jax.experimental import pallas as pl
from jax.experimental.pallas import tpu as pltpu
from jax.experimental.pallas import tpu_sc as plsc

NC = 2
NS = 16
NW = NC * NS


NBUF = 5


@functools.partial(jax.jit, static_argnames=("batch", "seq", "emb"))
def _embedding_gather(idx_t, table, *, batch, seq, emb):
    ipw = batch // NW
    n_rounds = seq // NBUF

    mesh = plsc.VectorSubcoreMesh(core_axis_name="c", subcore_axis_name="s")

    @functools.partial(
        pl.kernel,
        mesh=mesh,
        compiler_params=pltpu.CompilerParams(use_tc_tiling_on_sc=False),
        out_type=jax.ShapeDtypeStruct((seq, batch, emb), jnp.float32),
        scratch_types=[
            pltpu.VMEM((seq, ipw), jnp.int32),
        ] + [pltpu.VMEM((ipw, 2 * emb), jnp.float32)] * NBUF
          + [pltpu.SemaphoreType.DMA] * (2 * NBUF),
    )
    def k(table_hbm, idxt_hbm, out_hbm, idx_v, *bufs_and_sems):
        bufs = bufs_and_sems[:NBUF]
        gsem = bufs_and_sems[NBUF:2 * NBUF]
        wsem = bufs_and_sems[2 * NBUF:]
        wid = lax.axis_index("s") * NC + lax.axis_index("c")
        samples = pl.ds(wid * ipw, ipw)
        pltpu.sync_copy(idxt_hbm.at[:, samples], idx_v)

        for b in range(NBUF):
            pltpu.async_copy(table_hbm.at[idx_v.at[b]], bufs[b], gsem[b])

        emb_half = pl.ds(0, emb)

        def body(p, _):
            s0 = p * NBUF
            for b in range(NBUF):
                pltpu.make_async_copy(
                    table_hbm.at[idx_v.at[s0 + b]], bufs[b], gsem[b]).wait()
                pltpu.async_copy(bufs[b].at[:, emb_half],
                                 out_hbm.at[s0 + b, samples], wsem[b])
            for b in range(NBUF):
                pltpu.make_async_copy(
                    bufs[b].at[:, emb_half],
                    out_hbm.at[s0 + b, samples], wsem[b]).wait()
                pltpu.async_copy(
                    table_hbm.at[idx_v.at[s0 + NBUF + b]], bufs[b], gsem[b])
            return _

        lax.fori_loop(0, n_rounds - 1, body, None)

        sl = seq - NBUF
        for b in range(NBUF):
            pltpu.make_async_copy(
                table_hbm.at[idx_v.at[sl + b]], bufs[b], gsem[b]).wait()
            pltpu.async_copy(bufs[b].at[:, emb_half],
                             out_hbm.at[sl + b, samples], wsem[b])
        for b in range(NBUF):
            pltpu.make_async_copy(
                bufs[b].at[:, emb_half],
                out_hbm.at[sl + b, samples], wsem[b]).wait()

    return jnp.swapaxes(k(table, idx_t), 0, 1)


def kernel(input_tensor, embedding_table):
    batch, seq = input_tensor.shape
    _, emb = embedding_table.shape
    idx_t = input_tensor.T.astype(jnp.int32)
    table_p = jnp.pad(embedding_table, ((0, 0), (0, 128 - emb)))
    return _embedding_gather(idx_t, table_p,
                             batch=batch, seq=seq, emb=emb)


# speedup vs baseline: 1.6577x; 1.5624x over previous
import functools

import jax
import jax.numpy as jnp
from jax import lax
from jax.experimental import pallas as pl
from jax.experimental.pallas import tpu as pltpu
from jax.experimental.pallas import tpu_sc as plsc

NC = 2
NS = 16
NW = NC * NS


NBUF = 5
TBLK = 8192


def _tpose_block(tin_ref, out_ref):
    out_ref[:, 0:tin_ref.shape[0]] = tin_ref[...].T


def _transpose_pad(table_t, vocab, emb):
    return pl.pallas_call(
        _tpose_block,
        grid=(pl.cdiv(vocab, TBLK),),
        in_specs=[pl.BlockSpec((emb, TBLK), lambda i: (0, i))],
        out_specs=pl.BlockSpec((TBLK, 128), lambda i: (i, 0)),
        out_shape=jax.ShapeDtypeStruct((vocab, 128), jnp.float32),
    )(table_t)


@functools.partial(jax.jit, static_argnames=("batch", "seq", "emb"))
def _embedding_gather(idx_t, table, *, batch, seq, emb):
    ipw = batch // NW
    n_rounds = seq // NBUF

    mesh = plsc.VectorSubcoreMesh(core_axis_name="c", subcore_axis_name="s")

    @functools.partial(
        pl.kernel,
        mesh=mesh,
        compiler_params=pltpu.CompilerParams(use_tc_tiling_on_sc=False),
        out_type=jax.ShapeDtypeStruct((seq, batch, emb), jnp.float32),
        scratch_types=[
            pltpu.VMEM((seq, ipw), jnp.int32),
        ] + [pltpu.VMEM((ipw, 2 * emb), jnp.float32)] * NBUF
          + [pltpu.SemaphoreType.DMA] * (2 * NBUF),
    )
    def k(table_hbm, idxt_hbm, out_hbm, idx_v, *bufs_and_sems):
        bufs = bufs_and_sems[:NBUF]
        gsem = bufs_and_sems[NBUF:2 * NBUF]
        wsem = bufs_and_sems[2 * NBUF:]
        wid = lax.axis_index("s") * NC + lax.axis_index("c")
        samples = pl.ds(wid * ipw, ipw)
        pltpu.sync_copy(idxt_hbm.at[:, samples], idx_v)

        for b in range(NBUF):
            pltpu.async_copy(table_hbm.at[idx_v.at[b]], bufs[b], gsem[b])

        emb_half = pl.ds(0, emb)

        def body(p, _):
            s0 = p * NBUF
            for b in range(NBUF):
                pltpu.make_async_copy(
                    table_hbm.at[idx_v.at[s0 + b]], bufs[b], gsem[b]).wait()
                pltpu.async_copy(bufs[b].at[:, emb_half],
                                 out_hbm.at[s0 + b, samples], wsem[b])
            for b in range(NBUF):
                pltpu.make_async_copy(
                    bufs[b].at[:, emb_half],
                    out_hbm.at[s0 + b, samples], wsem[b]).wait()
                pltpu.async_copy(
                    table_hbm.at[idx_v.at[s0 + NBUF + b]], bufs[b], gsem[b])
            return _

        lax.fori_loop(0, n_rounds - 1, body, None)

        sl = seq - NBUF
        for b in range(NBUF):
            pltpu.make_async_copy(
                table_hbm.at[idx_v.at[sl + b]], bufs[b], gsem[b]).wait()
            pltpu.async_copy(bufs[b].at[:, emb_half],
                             out_hbm.at[sl + b, samples], wsem[b])
        for b in range(NBUF):
            pltpu.make_async_copy(
                bufs[b].at[:, emb_half],
                out_hbm.at[sl + b, samples], wsem[b]).wait()

    return jnp.swapaxes(k(table, idx_t), 0, 1)


def kernel(input_tensor, embedding_table):
    batch, seq = input_tensor.shape
    _, emb = embedding_table.shape
    idx_t = input_tensor.T.astype(jnp.int32)
    vocab = embedding_table.shape[0]
    table_p = _transpose_pad(embedding_table.T, vocab, emb)
    return _embedding_gather(idx_t, table_p,
                             batch=batch, seq=seq, emb=emb)
